# Initial kernel scaffold; baseline (speedup 1.0000x reference)
import jax
import jax.numpy as jnp
from jax.experimental import pallas as pl


def kernel(src_desc, src_kpts, tgt_desc, tgt_kpts):
    raise NotImplementedError("write your pallas kernel here")


# speedup vs baseline: 5.3397x; 5.3397x over previous
import functools

import jax
import jax.numpy as jnp
from jax import lax
from jax.experimental import pallas as pl
from jax.experimental.pallas import tpu as pltpu
from jax.experimental.pallas import tpu_sc as plsc

_MATCH_THRESHOLD = 0.3
_N = 8192
_M = 8192
_C = 64
_HW = 64
_NTILES = 32
_PPT = (_N + _M) // _NTILES
_PG = 16
_NG = _PPT // _PG


def _floor_f32(x):
    t = x.astype(jnp.int32)
    tf = t.astype(jnp.float32)
    adj = (tf > x).astype(jnp.int32)
    i = t - adj
    return i, i.astype(jnp.float32)


def _cubic_w(t):
    a = -0.75
    t1 = t + 1.0
    t3 = 1.0 - t
    t4 = 2.0 - t
    w0 = a * (t1 * t1 * t1) - 5.0 * a * (t1 * t1) + 8.0 * a * t1 - 4.0 * a
    w1 = (a + 2.0) * (t * t * t) - (a + 3.0) * (t * t) + 1.0
    w2 = (a + 2.0) * (t3 * t3 * t3) - (a + 3.0) * (t3 * t3) + 1.0
    w3 = a * (t4 * t4 * t4) - 5.0 * a * (t4 * t4) + 8.0 * a * t4 - 4.0 * a
    return (w0, w1, w2, w3)


def _sc_sample_body(table_hbm, kx_hbm, ky_hbm, out_hbm,
                    kxv, kyv, ibuf0, ibuf1, rows0, rows1, outt,
                    sem0, sem1):
    wid = lax.axis_index("s") * 2 + lax.axis_index("c")
    base = wid * _PPT
    toff = jnp.where(base >= _N, _HW * _HW, 0).astype(jnp.int32)

    pltpu.sync_copy(kx_hbm.at[pl.ds(base, _PPT)], kxv)
    pltpu.sync_copy(ky_hbm.at[pl.ds(base, _PPT)], kyv)

    iota = lax.iota(jnp.int32, 16)

    def group_body(g, carry):
        p0 = g * _PG
        kx = kxv[pl.ds(p0, _PG)]
        ky = kyv[pl.ds(p0, _PG)]
        kpx = kx / 8.0
        kpy = ky / 8.0
        gx = 2.0 * (kpx / (_HW - 1.0)) - 1.0
        gy = 2.0 * (kpy / (_HW - 1.0)) - 1.0
        ix = ((gx + 1.0) * _HW - 1.0) / 2.0
        iy = ((gy + 1.0) * _HW - 1.0) / 2.0
        ix0i, ix0f = _floor_f32(ix)
        iy0i, iy0f = _floor_f32(iy)
        tx = ix - ix0f
        ty = iy - iy0f
        wx = _cubic_w(tx)
        wy = _cubic_w(ty)

        xxc = []
        wxm = []
        for i in range(4):
            xx = ix0i + (i - 1)
            vx = ((xx >= 0) & (xx < _HW)).astype(jnp.float32)
            xxc.append(jnp.minimum(jnp.maximum(xx, 0), _HW - 1))
            wxm.append(wx[i] * vx)

        ws = []
        for j in range(4):
            yy = iy0i + (j - 1)
            vy = ((yy >= 0) & (yy < _HW)).astype(jnp.float32)
            yyc = jnp.minimum(jnp.maximum(yy, 0), _HW - 1)
            rowbase = toff + yyc * _HW
            wyj = wy[j] * vy
            for i in range(4):
                t = j * 4 + i
                idx = rowbase + xxc[i]
                if t < 8:
                    ibuf0[pl.ds(t * 16, 16)] = idx
                else:
                    ibuf1[pl.ds((t - 8) * 16, 16)] = idx
                ws.append(wyj * wxm[i])

        cp0 = pltpu.async_copy(table_hbm.at[ibuf0], rows0, sem0)
        cp1 = pltpu.async_copy(table_hbm.at[ibuf1], rows1, sem1)
        cp0.wait()
        cp1.wait()

        def chan_body(ch, cc):
            col = jnp.full((16,), ch, jnp.int32)
            acc = jnp.zeros((16,), jnp.float32)
            for t in range(16):
                rv = rows0 if t < 8 else rows1
                rowi = iota + (t % 8) * 16
                v = plsc.load_gather(rv, [rowi, col])
                acc = acc + ws[t] * v
            outt[ch, pl.ds(p0, _PG)] = acc
            return cc

        lax.fori_loop(0, _C, chan_body, 0)
        return carry

    lax.fori_loop(0, _NG, group_body, 0)
    pltpu.sync_copy(outt, out_hbm.at[wid])


def _sc_sample(table, kx, ky):
    mesh = plsc.VectorSubcoreMesh(core_axis_name="c", subcore_axis_name="s")
    f = pl.kernel(
        _sc_sample_body,
        out_type=jax.ShapeDtypeStruct((_NTILES, _C, _PPT), jnp.float32),
        mesh=mesh,
        compiler_params=pltpu.CompilerParams(needs_layout_passes=False,
                                             use_tc_tiling_on_sc=False),
        scratch_types=[
            pltpu.VMEM((_PPT,), jnp.float32),
            pltpu.VMEM((_PPT,), jnp.float32),
            pltpu.VMEM((128,), jnp.int32),
            pltpu.VMEM((128,), jnp.int32),
            pltpu.VMEM((128, _C), jnp.float32),
            pltpu.VMEM((128, _C), jnp.float32),
            pltpu.VMEM((_C, _PPT), jnp.float32),
            pltpu.SemaphoreType.DMA,
            pltpu.SemaphoreType.DMA,
        ],
    )
    return f(table, kx, ky)


_TR = 1024
_TCOL = 512


def _tc_body(fst_ref, ftt_ref, max1_ref, idx1_ref, idx2_ref,
             rmax, ridx, cmax, cidx):
    c = pl.program_id(0)
    r = pl.program_id(1)
    nc = pl.num_programs(0)
    nr = pl.num_programs(1)

    fs = fst_ref[...]
    ns = jnp.sqrt(jnp.sum(fs * fs, axis=0, keepdims=True))
    fsn = fs / jnp.maximum(ns, 1e-12)
    ft = ftt_ref[...]
    nt = jnp.sqrt(jnp.sum(ft * ft, axis=0, keepdims=True))
    ftn = ft / jnp.maximum(nt, 1e-12)

    s = lax.dot_general(fsn, ftn, (((0,), (0,)), ((), ())),
                        preferred_element_type=jnp.float32)

    big = jnp.int32(1 << 30)
    m = jnp.max(s, axis=1)
    colio = lax.broadcasted_iota(jnp.int32, s.shape, 1)
    a = jnp.min(jnp.where(s == m[:, None], colio, big), axis=1) + c * _TCOL

    cm = jnp.max(s, axis=0)
    rowio = lax.broadcasted_iota(jnp.int32, s.shape, 0)
    ci = jnp.min(jnp.where(s == cm[None, :], rowio, big), axis=0) + r * _TR

    rsl = pl.ds(r * _TR, _TR)

    @pl.when(c == 0)
    def _():
        rmax[rsl] = m
        ridx[rsl] = a

    @pl.when(c != 0)
    def _():
        pm = rmax[rsl]
        better = m > pm
        ridx[rsl] = jnp.where(better, a, ridx[rsl])
        rmax[rsl] = jnp.where(better, m, pm)

    @pl.when(r == 0)
    def _():
        cmax[...] = cm
        cidx[...] = ci

    @pl.when(r != 0)
    def _():
        pcm = cmax[...]
        cbetter = cm > pcm
        cidx[...] = jnp.where(cbetter, ci, cidx[...])
        cmax[...] = jnp.where(cbetter, cm, pcm)

    @pl.when(r == nr - 1)
    def _():
        idx2_ref[...] = cidx[...]

    @pl.when(c == nc - 1)
    def _():
        max1_ref[...] = rmax[rsl]
        idx1_ref[...] = ridx[rsl]


def _tc_simargmax(fst, ftt):
    ncol = _M // _TCOL
    nrow = _N // _TR
    return pl.pallas_call(
        _tc_body,
        grid=(ncol, nrow),
        in_specs=[
            pl.BlockSpec((_C, _TR), lambda c, r: (0, r)),
            pl.BlockSpec((_C, _TCOL), lambda c, r: (0, c)),
        ],
        out_specs=[
            pl.BlockSpec((_TR,), lambda c, r: (r,)),
            pl.BlockSpec((_TR,), lambda c, r: (r,)),
            pl.BlockSpec((_TCOL,), lambda c, r: (c,)),
        ],
        out_shape=[
            jax.ShapeDtypeStruct((_N,), jnp.float32),
            jax.ShapeDtypeStruct((_N,), jnp.int32),
            jax.ShapeDtypeStruct((_M,), jnp.int32),
        ],
        scratch_shapes=[
            pltpu.VMEM((_N,), jnp.float32),
            pltpu.VMEM((_N,), jnp.int32),
            pltpu.VMEM((_TCOL,), jnp.float32),
            pltpu.VMEM((_TCOL,), jnp.int32),
        ],
    )(fst, ftt)


_CPT = _N // _NTILES


def _sc_mutual_body(max1_hbm, idx1_hbm, idx2_hbm, scores_hbm, valid_hbm,
                    idx2v, i1v, m1v, sv, vv):
    wid = lax.axis_index("s") * 2 + lax.axis_index("c")
    base = wid * _CPT
    pltpu.sync_copy(idx2_hbm, idx2v)
    pltpu.sync_copy(idx1_hbm.at[pl.ds(base, _CPT)], i1v)
    pltpu.sync_copy(max1_hbm.at[pl.ds(base, _CPT)], m1v)
    iota = lax.iota(jnp.int32, 16)

    def body(g, carry):
        p0 = g * 16
        j = i1v[pl.ds(p0, 16)]
        back = plsc.load_gather(idx2v, [j])
        me = base + p0 + iota
        m1 = m1v[pl.ds(p0, 16)]
        ok = (back == me) & (m1 > _MATCH_THRESHOLD)
        sv[pl.ds(p0, 16)] = jnp.where(ok, m1, 0.0)
        vv[pl.ds(p0, 16)] = ok.astype(jnp.int32)
        return carry

    lax.fori_loop(0, _CPT // 16, body, 0)
    pltpu.sync_copy(sv, scores_hbm.at[pl.ds(base, _CPT)])
    pltpu.sync_copy(vv, valid_hbm.at[pl.ds(base, _CPT)])


def _sc_mutual(max1, idx1, idx2):
    mesh = plsc.VectorSubcoreMesh(core_axis_name="c", subcore_axis_name="s")
    f = pl.kernel(
        _sc_mutual_body,
        out_type=(jax.ShapeDtypeStruct((_N,), jnp.float32),
                  jax.ShapeDtypeStruct((_N,), jnp.int32)),
        mesh=mesh,
        compiler_params=pltpu.CompilerParams(needs_layout_passes=False),
        scratch_types=[
            pltpu.VMEM((_M,), jnp.int32),
            pltpu.VMEM((_CPT,), jnp.int32),
            pltpu.VMEM((_CPT,), jnp.float32),
            pltpu.VMEM((_CPT,), jnp.float32),
            pltpu.VMEM((_CPT,), jnp.int32),
        ],
    )
    return f(max1, idx1, idx2)


@jax.jit
def kernel(src_desc, src_kpts, tgt_desc, tgt_kpts):
    ts = jnp.transpose(src_desc[0], (1, 2, 0)).reshape(_HW * _HW, _C)
    tt = jnp.transpose(tgt_desc[0], (1, 2, 0)).reshape(_HW * _HW, _C)
    table = jnp.concatenate([ts, tt], axis=0)
    kx = jnp.concatenate([src_kpts[:, 0], tgt_kpts[:, 0]])
    ky = jnp.concatenate([src_kpts[:, 1], tgt_kpts[:, 1]])

    feats = _sc_sample(table, kx, ky)
    featst = jnp.transpose(feats, (1, 0, 2)).reshape(_C, _N + _M)
    fst = featst[:, :_N]
    ftt = featst[:, _N:]

    max1, idx1, idx2 = _tc_simargmax(fst, ftt)
    scores, valid = _sc_mutual(max1, idx1, idx2)
    return (scores, idx1, valid.astype(bool))


# speedup vs baseline: 15.3838x; 2.8810x over previous
import functools

import jax
import jax.numpy as jnp
from jax import lax
from jax.experimental import pallas as pl
from jax.experimental.pallas import tpu as pltpu
from jax.experimental.pallas import tpu_sc as plsc

_MATCH_THRESHOLD = 0.3
_N = 8192
_M = 8192
_C = 64
_HW = 64
_NTILES = 32
_PPT = (_N + _M) // _NTILES
_PG = 16
_NG = _PPT // _PG


def _floor_f32(x):
    t = x.astype(jnp.int32)
    tf = t.astype(jnp.float32)
    adj = (tf > x).astype(jnp.int32)
    i = t - adj
    return i, i.astype(jnp.float32)


def _cubic_w(t):
    a = -0.75
    t1 = t + 1.0
    t3 = 1.0 - t
    t4 = 2.0 - t
    w0 = a * (t1 * t1 * t1) - 5.0 * a * (t1 * t1) + 8.0 * a * t1 - 4.0 * a
    w1 = (a + 2.0) * (t * t * t) - (a + 3.0) * (t * t) + 1.0
    w2 = (a + 2.0) * (t3 * t3 * t3) - (a + 3.0) * (t3 * t3) + 1.0
    w3 = a * (t4 * t4 * t4) - 5.0 * a * (t4 * t4) + 8.0 * a * t4 - 4.0 * a
    return (w0, w1, w2, w3)


def _sc_sample_body(table_hbm, kx_hbm, ky_hbm, out_hbm,
                    kxv, kyv, ibufs, rows, wbufs, outt, sems):
    wid = lax.axis_index("s") * 2 + lax.axis_index("c")
    base = wid * _PPT
    toff = jnp.where(base >= _N, _HW * _HW, 0).astype(jnp.int32)

    pltpu.sync_copy(kx_hbm.at[pl.ds(base, _PPT)], kxv)
    pltpu.sync_copy(ky_hbm.at[pl.ds(base, _PPT)], kyv)

    iota = lax.iota(jnp.int32, 16)

    def compute_iw(g, b):
        p0 = g * _PG
        kx = kxv[pl.ds(p0, _PG)]
        ky = kyv[pl.ds(p0, _PG)]
        kpx = kx / 8.0
        kpy = ky / 8.0
        gx = 2.0 * (kpx / (_HW - 1.0)) - 1.0
        gy = 2.0 * (kpy / (_HW - 1.0)) - 1.0
        ix = ((gx + 1.0) * _HW - 1.0) / 2.0
        iy = ((gy + 1.0) * _HW - 1.0) / 2.0
        ix0i, ix0f = _floor_f32(ix)
        iy0i, iy0f = _floor_f32(iy)
        tx = ix - ix0f
        ty = iy - iy0f
        wx = _cubic_w(tx)
        wy = _cubic_w(ty)

        xxc = []
        wxm = []
        for i in range(4):
            xx = ix0i + (i - 1)
            vx = ((xx >= 0) & (xx < _HW)).astype(jnp.float32)
            xxc.append(jnp.minimum(jnp.maximum(xx, 0), _HW - 1))
            wxm.append(wx[i] * vx)

        for j in range(4):
            yy = iy0i + (j - 1)
            vy = ((yy >= 0) & (yy < _HW)).astype(jnp.float32)
            yyc = jnp.minimum(jnp.maximum(yy, 0), _HW - 1)
            rowbase = toff + yyc * _HW
            wyj = wy[j] * vy
            for i in range(4):
                t = j * 4 + i
                half = 2 * b + (0 if t < 8 else 1)
                ibufs[half, pl.ds((t % 8) * 16, 16)] = rowbase + xxc[i]
                plsc.store_scatter(wbufs,
                                   [jnp.full((16,), b, jnp.int32),
                                    iota * 16 + t],
                                   wyj * wxm[i])

    def start_dma(b):
        pltpu.async_copy(table_hbm.at[ibufs.at[2 * b]],
                         rows.at[2 * b], sems[2 * b])
        pltpu.async_copy(table_hbm.at[ibufs.at[2 * b + 1]],
                         rows.at[2 * b + 1], sems[2 * b + 1])

    def wait_dma(b):
        pltpu.make_async_copy(table_hbm.at[ibufs.at[2 * b]],
                              rows.at[2 * b], sems[2 * b]).wait()
        pltpu.make_async_copy(table_hbm.at[ibufs.at[2 * b + 1]],
                              rows.at[2 * b + 1], sems[2 * b + 1]).wait()

    def accum(g, b):
        p0 = g * _PG
        r0 = rows.at[2 * b]
        r1 = rows.at[2 * b + 1]

        def point_body(p, cc):
            accs = [jnp.zeros((16,), jnp.float32) for _ in range(4)]
            wv = wbufs[b, pl.ds(p * 16, 16)]
            for t in range(16):
                rv = r0 if t < 8 else r1
                ri = (t % 8) * 16 + p
                w = wv[t]
                for q in range(4):
                    v = rv[ri, pl.ds(q * 16, 16)]
                    accs[q] = accs[q] + w * v
            for q in range(4):
                outt[p0 + p, pl.ds(q * 16, 16)] = accs[q]
            return cc

        lax.fori_loop(0, _PG, point_body, 0)

    compute_iw(0, 0)
    start_dma(0)

    def pair_body(k, carry):
        g = 2 * k
        compute_iw(g + 1, 1)
        start_dma(1)
        wait_dma(0)
        accum(g, 0)

        @pl.when(k < _NG // 2 - 1)
        def _():
            compute_iw(g + 2, 0)
            start_dma(0)

        wait_dma(1)
        accum(g + 1, 1)
        return carry

    lax.fori_loop(0, _NG // 2, pair_body, 0)
    pltpu.sync_copy(outt, out_hbm.at[wid])


def _sc_sample(table, kx, ky):
    mesh = plsc.VectorSubcoreMesh(core_axis_name="c", subcore_axis_name="s")
    f = pl.kernel(
        _sc_sample_body,
        out_type=jax.ShapeDtypeStruct((_NTILES, _PPT, _C), jnp.float32),
        mesh=mesh,
        compiler_params=pltpu.CompilerParams(needs_layout_passes=False,
                                             use_tc_tiling_on_sc=False),
        scratch_types=[
            pltpu.VMEM((_PPT,), jnp.float32),
            pltpu.VMEM((_PPT,), jnp.float32),
            pltpu.VMEM((4, 128), jnp.int32),
            pltpu.VMEM((4, 128, _C), jnp.float32),
            pltpu.VMEM((2, 256), jnp.float32),
            pltpu.VMEM((_PPT, _C), jnp.float32),
            [pltpu.SemaphoreType.DMA] * 4,
        ],
    )
    return f(table, kx, ky)


_TR = 2048
_TCOL = 1024


def _tc_body(fst_ref, ftt_ref, max1_ref, idx1_ref, idx2_ref,
             rmax, ridx, cmax, cidx, fsnt, ftnt):
    c = pl.program_id(0)
    r = pl.program_id(1)
    nc = pl.num_programs(0)
    nr = pl.num_programs(1)

    rsl2 = pl.ds(r * _TR, _TR)
    csl2 = pl.ds(c * _TCOL, _TCOL)

    @pl.when(c == 0)
    def _():
        fs = fst_ref[rsl2, :].T
        ns = jnp.sqrt(jnp.sum(fs * fs, axis=0, keepdims=True))
        fsnt[:, rsl2] = fs / jnp.maximum(ns, 1e-12)

    @pl.when(r == 0)
    def _():
        ft = ftt_ref[csl2, :].T
        nt = jnp.sqrt(jnp.sum(ft * ft, axis=0, keepdims=True))
        ftnt[:, csl2] = ft / jnp.maximum(nt, 1e-12)

    s = lax.dot_general(fsnt[:, rsl2], ftnt[:, csl2], (((0,), (0,)), ((), ())),
                        preferred_element_type=jnp.float32)
    s2 = lax.dot_general(ftnt[:, csl2], fsnt[:, rsl2], (((0,), (0,)), ((), ())),
                         preferred_element_type=jnp.float32)

    big = jnp.int32(1 << 30)
    m = jnp.max(s2, axis=0)
    io2 = lax.broadcasted_iota(jnp.int32, s2.shape, 0)
    a = jnp.min(jnp.where(s2 == m[None, :], io2, big), axis=0) + c * _TCOL

    cm = jnp.max(s, axis=0)
    io = lax.broadcasted_iota(jnp.int32, s.shape, 0)
    ci = jnp.min(jnp.where(s == cm[None, :], io, big), axis=0) + r * _TR

    rsl = pl.ds(r * _TR, _TR)

    @pl.when(c == 0)
    def _():
        rmax[rsl] = m
        ridx[rsl] = a

    @pl.when(c != 0)
    def _():
        pm = rmax[rsl]
        better = m > pm
        ridx[rsl] = jnp.where(better, a, ridx[rsl])
        rmax[rsl] = jnp.where(better, m, pm)

    @pl.when(r == 0)
    def _():
        cmax[...] = cm
        cidx[...] = ci

    @pl.when(r != 0)
    def _():
        pcm = cmax[...]
        cbetter = cm > pcm
        cidx[...] = jnp.where(cbetter, ci, cidx[...])
        cmax[...] = jnp.where(cbetter, cm, pcm)

    @pl.when(r == nr - 1)
    def _():
        idx2_ref[...] = cidx[...]

    @pl.when(c == nc - 1)
    def _():
        max1_ref[...] = rmax[rsl]
        idx1_ref[...] = ridx[rsl]


def _tc_simargmax(fst, ftt):
    ncol = _M // _TCOL
    nrow = _N // _TR
    return pl.pallas_call(
        _tc_body,
        grid=(ncol, nrow),
        in_specs=[
            pl.BlockSpec((_N, _C), lambda c, r: (0, 0)),
            pl.BlockSpec((_M, _C), lambda c, r: (0, 0)),
        ],
        out_specs=[
            pl.BlockSpec((_TR,), lambda c, r: (r,)),
            pl.BlockSpec((_TR,), lambda c, r: (r,)),
            pl.BlockSpec((_TCOL,), lambda c, r: (c,)),
        ],
        out_shape=[
            jax.ShapeDtypeStruct((_N,), jnp.float32),
            jax.ShapeDtypeStruct((_N,), jnp.int32),
            jax.ShapeDtypeStruct((_M,), jnp.int32),
        ],
        scratch_shapes=[
            pltpu.VMEM((_N,), jnp.float32),
            pltpu.VMEM((_N,), jnp.int32),
            pltpu.VMEM((_TCOL,), jnp.float32),
            pltpu.VMEM((_TCOL,), jnp.int32),
            pltpu.VMEM((_C, _N), jnp.float32),
            pltpu.VMEM((_C, _M), jnp.float32),
        ],
    )(fst, ftt)


_CPT = _N // _NTILES


def _sc_mutual_body(max1_hbm, idx1_hbm, idx2_hbm, scores_hbm, valid_hbm,
                    idx2v, i1v, m1v, sv, vv):
    wid = lax.axis_index("s") * 2 + lax.axis_index("c")
    base = wid * _CPT
    pltpu.sync_copy(idx2_hbm, idx2v)
    pltpu.sync_copy(idx1_hbm.at[pl.ds(base, _CPT)], i1v)
    pltpu.sync_copy(max1_hbm.at[pl.ds(base, _CPT)], m1v)
    iota = lax.iota(jnp.int32, 16)

    def body(g, carry):
        p0 = g * 16
        j = i1v[pl.ds(p0, 16)]
        back = plsc.load_gather(idx2v, [j])
        me = base + p0 + iota
        m1 = m1v[pl.ds(p0, 16)]
        ok = (back == me) & (m1 > _MATCH_THRESHOLD)
        sv[pl.ds(p0, 16)] = jnp.where(ok, m1, 0.0)
        vv[pl.ds(p0, 16)] = ok.astype(jnp.int32)
        return carry

    lax.fori_loop(0, _CPT // 16, body, 0)
    pltpu.sync_copy(sv, scores_hbm.at[pl.ds(base, _CPT)])
    pltpu.sync_copy(vv, valid_hbm.at[pl.ds(base, _CPT)])


def _sc_mutual(max1, idx1, idx2):
    mesh = plsc.VectorSubcoreMesh(core_axis_name="c", subcore_axis_name="s")
    f = pl.kernel(
        _sc_mutual_body,
        out_type=(jax.ShapeDtypeStruct((_N,), jnp.float32),
                  jax.ShapeDtypeStruct((_N,), jnp.int32)),
        mesh=mesh,
        compiler_params=pltpu.CompilerParams(needs_layout_passes=False),
        scratch_types=[
            pltpu.VMEM((_M,), jnp.int32),
            pltpu.VMEM((_CPT,), jnp.int32),
            pltpu.VMEM((_CPT,), jnp.float32),
            pltpu.VMEM((_CPT,), jnp.float32),
            pltpu.VMEM((_CPT,), jnp.int32),
        ],
    )
    return f(max1, idx1, idx2)


@jax.jit
def kernel(src_desc, src_kpts, tgt_desc, tgt_kpts):
    ts = jnp.transpose(src_desc[0], (1, 2, 0)).reshape(_HW * _HW, _C)
    tt = jnp.transpose(tgt_desc[0], (1, 2, 0)).reshape(_HW * _HW, _C)
    table = jnp.concatenate([ts, tt], axis=0)
    kx = jnp.concatenate([src_kpts[:, 0], tgt_kpts[:, 0]])
    ky = jnp.concatenate([src_kpts[:, 1], tgt_kpts[:, 1]])

    feats = _sc_sample(table, kx, ky).reshape(_N + _M, _C)
    fs = feats[:_N]
    ft = feats[_N:]

    max1, idx1, idx2 = _tc_simargmax(fs, ft)
    scores, valid = _sc_mutual(max1, idx1, idx2)
    return (scores, idx1, valid.astype(bool))


# speedup vs baseline: 15.9511x; 1.0369x over previous
import functools

import jax
import jax.numpy as jnp
from jax import lax
from jax.experimental import pallas as pl
from jax.experimental.pallas import tpu as pltpu
from jax.experimental.pallas import tpu_sc as plsc

_MATCH_THRESHOLD = 0.3
_N = 8192
_M = 8192
_C = 64
_HW = 64
_NTILES = 32
_PPT = (_N + _M) // _NTILES
_PG = 16
_NG = _PPT // _PG


def _floor_f32(x):
    t = x.astype(jnp.int32)
    tf = t.astype(jnp.float32)
    adj = (tf > x).astype(jnp.int32)
    i = t - adj
    return i, i.astype(jnp.float32)


def _cubic_w(t):
    a = -0.75
    t1 = t + 1.0
    t3 = 1.0 - t
    t4 = 2.0 - t
    w0 = a * (t1 * t1 * t1) - 5.0 * a * (t1 * t1) + 8.0 * a * t1 - 4.0 * a
    w1 = (a + 2.0) * (t * t * t) - (a + 3.0) * (t * t) + 1.0
    w2 = (a + 2.0) * (t3 * t3 * t3) - (a + 3.0) * (t3 * t3) + 1.0
    w3 = a * (t4 * t4 * t4) - 5.0 * a * (t4 * t4) + 8.0 * a * t4 - 4.0 * a
    return (w0, w1, w2, w3)


def _sc_sample_body(table_hbm, kx_hbm, ky_hbm, out_hbm,
                    kxv, kyv, ibufs, rows, wbufs, outt, sems):
    wid = lax.axis_index("s") * 2 + lax.axis_index("c")
    base = wid * _PPT
    toff = jnp.where(base >= _N, _HW * _HW, 0).astype(jnp.int32)

    pltpu.sync_copy(kx_hbm.at[pl.ds(base, _PPT)], kxv)
    pltpu.sync_copy(ky_hbm.at[pl.ds(base, _PPT)], kyv)

    iota = lax.iota(jnp.int32, 16)

    def compute_iw(g, b):
        p0 = g * _PG
        kx = kxv[pl.ds(p0, _PG)]
        ky = kyv[pl.ds(p0, _PG)]
        kpx = kx / 8.0
        kpy = ky / 8.0
        gx = 2.0 * (kpx / (_HW - 1.0)) - 1.0
        gy = 2.0 * (kpy / (_HW - 1.0)) - 1.0
        ix = ((gx + 1.0) * _HW - 1.0) / 2.0
        iy = ((gy + 1.0) * _HW - 1.0) / 2.0
        ix0i, ix0f = _floor_f32(ix)
        iy0i, iy0f = _floor_f32(iy)
        tx = ix - ix0f
        ty = iy - iy0f
        wx = _cubic_w(tx)
        wy = _cubic_w(ty)

        xxc = []
        wxm = []
        for i in range(4):
            xx = ix0i + (i - 1)
            vx = ((xx >= 0) & (xx < _HW)).astype(jnp.float32)
            xxc.append(jnp.minimum(jnp.maximum(xx, 0), _HW - 1))
            wxm.append(wx[i] * vx)

        for j in range(4):
            yy = iy0i + (j - 1)
            vy = ((yy >= 0) & (yy < _HW)).astype(jnp.float32)
            yyc = jnp.minimum(jnp.maximum(yy, 0), _HW - 1)
            rowbase = toff + yyc * _HW
            wyj = wy[j] * vy
            for i in range(4):
                t = j * 4 + i
                half = 2 * b + (0 if t < 8 else 1)
                ibufs[half, pl.ds((t % 8) * 16, 16)] = rowbase + xxc[i]
                plsc.store_scatter(wbufs,
                                   [jnp.full((16,), b, jnp.int32),
                                    iota * 16 + t],
                                   wyj * wxm[i])

    def start_dma(b):
        pltpu.async_copy(table_hbm.at[ibufs.at[2 * b]],
                         rows.at[2 * b], sems[2 * b])
        pltpu.async_copy(table_hbm.at[ibufs.at[2 * b + 1]],
                         rows.at[2 * b + 1], sems[2 * b + 1])

    def wait_dma(b):
        pltpu.make_async_copy(table_hbm.at[ibufs.at[2 * b]],
                              rows.at[2 * b], sems[2 * b]).wait()
        pltpu.make_async_copy(table_hbm.at[ibufs.at[2 * b + 1]],
                              rows.at[2 * b + 1], sems[2 * b + 1]).wait()

    def accum(g, b):
        p0 = g * _PG
        r0 = rows.at[2 * b]
        r1 = rows.at[2 * b + 1]

        def point_body(p, cc):
            accs = [jnp.zeros((16,), jnp.float32) for _ in range(4)]
            wv = wbufs[b, pl.ds(p * 16, 16)]
            for t in range(16):
                rv = r0 if t < 8 else r1
                ri = (t % 8) * 16 + p
                w = wv[t]
                for q in range(4):
                    v = rv[ri, pl.ds(q * 16, 16)]
                    accs[q] = accs[q] + w * v
            for q in range(4):
                outt[p0 + p, pl.ds(q * 16, 16)] = accs[q]
            return cc

        lax.fori_loop(0, _PG, point_body, 0)

    compute_iw(0, 0)
    start_dma(0)

    def pair_body(k, carry):
        g = 2 * k
        compute_iw(g + 1, 1)
        start_dma(1)
        wait_dma(0)
        accum(g, 0)

        @pl.when(k < _NG // 2 - 1)
        def _():
            compute_iw(g + 2, 0)
            start_dma(0)

        wait_dma(1)
        accum(g + 1, 1)
        return carry

    lax.fori_loop(0, _NG // 2, pair_body, 0)
    pltpu.sync_copy(outt, out_hbm.at[wid])


def _sc_sample(table, kx, ky):
    mesh = plsc.VectorSubcoreMesh(core_axis_name="c", subcore_axis_name="s")
    f = pl.kernel(
        _sc_sample_body,
        out_type=jax.ShapeDtypeStruct((_NTILES, _PPT, _C), jnp.float32),
        mesh=mesh,
        compiler_params=pltpu.CompilerParams(needs_layout_passes=False,
                                             use_tc_tiling_on_sc=False),
        scratch_types=[
            pltpu.VMEM((_PPT,), jnp.float32),
            pltpu.VMEM((_PPT,), jnp.float32),
            pltpu.VMEM((4, 128), jnp.int32),
            pltpu.VMEM((4, 128, _C), jnp.float32),
            pltpu.VMEM((2, 256), jnp.float32),
            pltpu.VMEM((_PPT, _C), jnp.float32),
            [pltpu.SemaphoreType.DMA] * 4,
        ],
    )
    return f(table, kx, ky)


_TR = 2048
_TCOL = 2048


def _tc_body(fst_ref, ftt_ref, max1_ref, idx1_ref, idx2_ref,
             rmax, ridx, cmax, cidx, fsnt, ftnt):
    c = pl.program_id(0)
    r = pl.program_id(1)
    nc = pl.num_programs(0)
    nr = pl.num_programs(1)

    rsl2 = pl.ds(r * _TR, _TR)
    csl2 = pl.ds(c * _TCOL, _TCOL)

    @pl.when(c == 0)
    def _():
        fs = fst_ref[rsl2, :].T
        ns = jnp.sqrt(jnp.sum(fs * fs, axis=0, keepdims=True))
        fsnt[:, rsl2] = fs / jnp.maximum(ns, 1e-12)

    @pl.when(r == 0)
    def _():
        ft = ftt_ref[csl2, :].T
        nt = jnp.sqrt(jnp.sum(ft * ft, axis=0, keepdims=True))
        ftnt[:, csl2] = ft / jnp.maximum(nt, 1e-12)

    s = lax.dot_general(fsnt[:, rsl2], ftnt[:, csl2], (((0,), (0,)), ((), ())),
                        preferred_element_type=jnp.float32)
    s2 = lax.dot_general(ftnt[:, csl2], fsnt[:, rsl2], (((0,), (0,)), ((), ())),
                         preferred_element_type=jnp.float32)

    big = jnp.int32(1 << 30)
    m = jnp.max(s2, axis=0)
    io2 = lax.broadcasted_iota(jnp.int32, s2.shape, 0)
    a = jnp.min(jnp.where(s2 == m[None, :], io2, big), axis=0) + c * _TCOL

    cm = jnp.max(s, axis=0)
    io = lax.broadcasted_iota(jnp.int32, s.shape, 0)
    ci = jnp.min(jnp.where(s == cm[None, :], io, big), axis=0) + r * _TR

    rsl = pl.ds(r * _TR, _TR)

    @pl.when(c == 0)
    def _():
        rmax[rsl] = m
        ridx[rsl] = a

    @pl.when(c != 0)
    def _():
        pm = rmax[rsl]
        better = m > pm
        ridx[rsl] = jnp.where(better, a, ridx[rsl])
        rmax[rsl] = jnp.where(better, m, pm)

    @pl.when(r == 0)
    def _():
        cmax[...] = cm
        cidx[...] = ci

    @pl.when(r != 0)
    def _():
        pcm = cmax[...]
        cbetter = cm > pcm
        cidx[...] = jnp.where(cbetter, ci, cidx[...])
        cmax[...] = jnp.where(cbetter, cm, pcm)

    @pl.when(r == nr - 1)
    def _():
        idx2_ref[...] = cidx[...]

    @pl.when(c == nc - 1)
    def _():
        max1_ref[...] = rmax[rsl]
        idx1_ref[...] = ridx[rsl]


def _tc_simargmax(fst, ftt):
    ncol = _M // _TCOL
    nrow = _N // _TR
    return pl.pallas_call(
        _tc_body,
        grid=(ncol, nrow),
        in_specs=[
            pl.BlockSpec((_N, _C), lambda c, r: (0, 0)),
            pl.BlockSpec((_M, _C), lambda c, r: (0, 0)),
        ],
        out_specs=[
            pl.BlockSpec((_TR,), lambda c, r: (r,)),
            pl.BlockSpec((_TR,), lambda c, r: (r,)),
            pl.BlockSpec((_TCOL,), lambda c, r: (c,)),
        ],
        out_shape=[
            jax.ShapeDtypeStruct((_N,), jnp.float32),
            jax.ShapeDtypeStruct((_N,), jnp.int32),
            jax.ShapeDtypeStruct((_M,), jnp.int32),
        ],
        scratch_shapes=[
            pltpu.VMEM((_N,), jnp.float32),
            pltpu.VMEM((_N,), jnp.int32),
            pltpu.VMEM((_TCOL,), jnp.float32),
            pltpu.VMEM((_TCOL,), jnp.int32),
            pltpu.VMEM((_C, _N), jnp.float32),
            pltpu.VMEM((_C, _M), jnp.float32),
        ],
    )(fst, ftt)


_CPT = _N // _NTILES


def _sc_mutual_body(max1_hbm, idx1_hbm, idx2_hbm, scores_hbm, valid_hbm,
                    idx2v, i1v, m1v, sv, vv):
    wid = lax.axis_index("s") * 2 + lax.axis_index("c")
    base = wid * _CPT
    pltpu.sync_copy(idx2_hbm, idx2v)
    pltpu.sync_copy(idx1_hbm.at[pl.ds(base, _CPT)], i1v)
    pltpu.sync_copy(max1_hbm.at[pl.ds(base, _CPT)], m1v)
    iota = lax.iota(jnp.int32, 16)

    def body(g, carry):
        p0 = g * 16
        j = i1v[pl.ds(p0, 16)]
        back = plsc.load_gather(idx2v, [j])
        me = base + p0 + iota
        m1 = m1v[pl.ds(p0, 16)]
        ok = (back == me) & (m1 > _MATCH_THRESHOLD)
        sv[pl.ds(p0, 16)] = jnp.where(ok, m1, 0.0)
        vv[pl.ds(p0, 16)] = ok.astype(jnp.int32)
        return carry

    lax.fori_loop(0, _CPT // 16, body, 0)
    pltpu.sync_copy(sv, scores_hbm.at[pl.ds(base, _CPT)])
    pltpu.sync_copy(vv, valid_hbm.at[pl.ds(base, _CPT)])


def _sc_mutual(max1, idx1, idx2):
    mesh = plsc.VectorSubcoreMesh(core_axis_name="c", subcore_axis_name="s")
    f = pl.kernel(
        _sc_mutual_body,
        out_type=(jax.ShapeDtypeStruct((_N,), jnp.float32),
                  jax.ShapeDtypeStruct((_N,), jnp.int32)),
        mesh=mesh,
        compiler_params=pltpu.CompilerParams(needs_layout_passes=False),
        scratch_types=[
            pltpu.VMEM((_M,), jnp.int32),
            pltpu.VMEM((_CPT,), jnp.int32),
            pltpu.VMEM((_CPT,), jnp.float32),
            pltpu.VMEM((_CPT,), jnp.float32),
            pltpu.VMEM((_CPT,), jnp.int32),
        ],
    )
    return f(max1, idx1, idx2)


@jax.jit
def kernel(src_desc, src_kpts, tgt_desc, tgt_kpts):
    ts = jnp.transpose(src_desc[0], (1, 2, 0)).reshape(_HW * _HW, _C)
    tt = jnp.transpose(tgt_desc[0], (1, 2, 0)).reshape(_HW * _HW, _C)
    table = jnp.concatenate([ts, tt], axis=0)
    kx = jnp.concatenate([src_kpts[:, 0], tgt_kpts[:, 0]])
    ky = jnp.concatenate([src_kpts[:, 1], tgt_kpts[:, 1]])

    feats = _sc_sample(table, kx, ky).reshape(_N + _M, _C)
    fs = feats[:_N]
    ft = feats[_N:]

    max1, idx1, idx2 = _tc_simargmax(fs, ft)
    scores, valid = _sc_mutual(max1, idx1, idx2)
    return (scores, idx1, valid.astype(bool))
